# Initial kernel scaffold; baseline (speedup 1.0000x reference)
import jax
import jax.numpy as jnp
from jax.experimental import pallas as pl


def kernel(node_feats, pos, batch, W1, b1, W2, b2):
    raise NotImplementedError("write your pallas kernel here")


# speedup vs baseline: 3.3876x; 3.3876x over previous
import functools

import jax
import jax.numpy as jnp
from jax.experimental import pallas as pl

_G = 512
_TILE = 2048


def _fused_body(nf, pos, bat, w1, b1, w2, b2, q_out, seg_out, *, n_rows, tile):
    i = pl.program_id(0)

    @pl.when(i == 0)
    def _init():
        seg_out[...] = jnp.zeros_like(seg_out)

    x = nf[...]
    h = jnp.dot(x, w1[...], preferred_element_type=jnp.float32) + b1[...]
    h = h * jax.nn.sigmoid(h)
    q = jnp.dot(h, w2[...], preferred_element_type=jnp.float32) + b2[...]
    q_out[...] = q

    ridx = i * tile + jax.lax.broadcasted_iota(jnp.int32, (tile, 1), 0)
    validb = ridx < n_rows
    valid = validb.astype(jnp.float32)
    qv = jnp.where(validb, q, 0.0)
    wp = jnp.where(validb, pos[...], 0.0) * qv
    p = jnp.concatenate([wp, qv, valid], axis=1)

    b = bat[...]
    oh = (jax.lax.broadcasted_iota(jnp.int32, (_G, tile), 0) == b[None, :])
    seg_out[...] += jnp.dot(oh.astype(jnp.float32), p,
                            preferred_element_type=jnp.float32)


def kernel(node_feats, pos, batch, W1, b1, W2, b2):
    n, d = node_feats.shape
    h_dim = W1.shape[1]
    tile = _TILE
    grid = pl.cdiv(n, tile)

    q2, seg = pl.pallas_call(
        functools.partial(_fused_body, n_rows=n, tile=tile),
        grid=(grid,),
        in_specs=[
            pl.BlockSpec((tile, d), lambda i: (i, 0)),
            pl.BlockSpec((tile, 3), lambda i: (i, 0)),
            pl.BlockSpec((tile,), lambda i: (i,)),
            pl.BlockSpec((d, h_dim), lambda i: (0, 0)),
            pl.BlockSpec((1, h_dim), lambda i: (0, 0)),
            pl.BlockSpec((h_dim, 1), lambda i: (0, 0)),
            pl.BlockSpec((1, 1), lambda i: (0, 0)),
        ],
        out_specs=[
            pl.BlockSpec((tile, 1), lambda i: (i, 0)),
            pl.BlockSpec((_G, 5), lambda i: (0, 0)),
        ],
        out_shape=[
            jax.ShapeDtypeStruct((n, 1), jnp.float32),
            jax.ShapeDtypeStruct((_G, 5), jnp.float32),
        ],
    )(node_feats, pos, batch, W1, b1.reshape(1, h_dim), W2, b2.reshape(1, 1))

    charges = q2.reshape(n)
    dipole = seg[:, 0:3]
    total_charge = seg[:, 3]
    num_atoms = seg[:, 4]
    return (dipole, charges, total_charge, num_atoms)
